# Initial kernel scaffold; baseline (speedup 1.0000x reference)
import jax
import jax.numpy as jnp
from jax.experimental import pallas as pl


def kernel(x, edge_index, W0a, W1a, ba, W0b, W1b, bb, Wo, bo):
    raise NotImplementedError("write your pallas kernel here")


# speedup vs baseline: 25.7050x; 25.7050x over previous
import functools

import jax
import jax.numpy as jnp
from jax import lax
from jax.experimental import pallas as pl
from jax.experimental.pallas import tpu as pltpu
from jax.experimental.pallas import tpu_sc as plsc

N = 10000
E = 320000
D = 128
HP = 16
NC = 2
NS = 16
NW = NC * NS
EW = E // NW
G = 100
K = EW // G
RT = N // NS
RTC = 632


def _tile_row_base(s):
    return pl.multiple_of((s * RT) // 8 * 8, 8)

_mesh = plsc.VectorSubcoreMesh(core_axis_name="c", subcore_axis_name="s")


@functools.partial(
    pl.kernel,
    out_type=jax.ShapeDtypeStruct((NC, N, HP), jnp.float32),
    mesh=_mesh,
    compiler_params=pltpu.CompilerParams(use_tc_tiling_on_sc=False),
    scratch_types=[
        pltpu.VMEM((G, K), jnp.int32),
        pltpu.VMEM((K, HP), jnp.float32),
        pltpu.VMEM((RTC, HP), jnp.float32),
        pltpu.VMEM_SHARED((N, HP), jnp.float32),
    ],
)
def _sc_degree(src_hbm, deg_out, src_v, ones_v, zbuf, acc):
    c = lax.axis_index("c")
    s = lax.axis_index("s")
    wid = s * NC + c

    pltpu.sync_copy(src_hbm.at[wid], src_v)

    def _ones_row(i, carry):
        ones_v[i] = jnp.ones((HP,), jnp.float32)
        return carry

    lax.fori_loop(0, K, _ones_row, 0)

    def _zero_row(i, carry):
        zbuf[i] = jnp.zeros((HP,), jnp.float32)
        return carry

    lax.fori_loop(0, RTC, _zero_row, 0)
    base = _tile_row_base(s)
    pltpu.sync_copy(zbuf, acc.at[pl.ds(base, RTC)])
    plsc.subcore_barrier()

    def _scat(g, carry):
        pltpu.sync_copy(ones_v, acc.at[src_v.at[g]], add=True)
        return carry

    lax.fori_loop(0, G, _scat, 0)
    plsc.subcore_barrier()
    pltpu.sync_copy(acc.at[pl.ds(base, RTC)], deg_out.at[c, pl.ds(base, RTC)])


@functools.partial(
    pl.kernel,
    out_type=jax.ShapeDtypeStruct((NC, N, HP), jnp.float32),
    mesh=_mesh,
    compiler_params=pltpu.CompilerParams(use_tc_tiling_on_sc=False),
    scratch_types=[
        pltpu.VMEM((G, K), jnp.int32),
        pltpu.VMEM((G, K), jnp.int32),
        pltpu.VMEM((K, HP), jnp.float32),
        pltpu.VMEM((RTC, HP), jnp.float32),
        pltpu.VMEM_SHARED((N, HP), jnp.float32),
        pltpu.SemaphoreType.DMA,
    ],
)
def _sc_gather_scatter(table_hbm, src_hbm, dst_hbm, out_hbm,
                       src_v, dst_v, rows_v, zbuf, acc, sem):
    c = lax.axis_index("c")
    s = lax.axis_index("s")
    wid = s * NC + c

    pltpu.sync_copy(src_hbm.at[wid], src_v)
    pltpu.sync_copy(dst_hbm.at[wid], dst_v)

    def _zero_row(i, carry):
        zbuf[i] = jnp.zeros((HP,), jnp.float32)
        return carry

    lax.fori_loop(0, RTC, _zero_row, 0)
    base = _tile_row_base(s)
    pltpu.sync_copy(zbuf, acc.at[pl.ds(base, RTC)])
    plsc.subcore_barrier()

    def _edge_group(g, carry):
        pltpu.async_copy(table_hbm.at[src_v.at[g]], rows_v, sem).wait()
        pltpu.sync_copy(rows_v, acc.at[dst_v.at[g]], add=True)
        return carry

    lax.fori_loop(0, G, _edge_group, 0)
    plsc.subcore_barrier()
    pltpu.sync_copy(acc.at[pl.ds(base, RTC)], out_hbm.at[c, pl.ds(base, RTC)])


def _row_spec(r):
    return pl.BlockSpec((r, HP), lambda i: (i, 0))


def _full_spec(shape):
    return pl.BlockSpec(shape, lambda i: tuple(0 for _ in shape))


_R = 2000


def _tc1(x, d0, d1, w1p, w0p):
    def body(x_ref, d0_ref, d1_ref, w1_ref, w0_ref, ta_ref, za_ref, di_ref):
        deg = d0_ref[:, 0:1] + d1_ref[:, 0:1]
        dinv = jnp.where(deg > 0, lax.rsqrt(jnp.maximum(deg, 1e-12)), 0.0)
        xb = x_ref[...]
        ta_ref[...] = dinv * jnp.dot(xb, w1_ref[...],
                                     preferred_element_type=jnp.float32)
        za_ref[...] = jnp.dot(xb, w0_ref[...],
                              preferred_element_type=jnp.float32)
        di_ref[...] = jnp.broadcast_to(dinv, (_R, HP))

    return pl.pallas_call(
        body,
        grid=(N // _R,),
        in_specs=[
            pl.BlockSpec((_R, D), lambda i: (i, 0)),
            _row_spec(_R), _row_spec(_R),
            _full_spec((D, HP)), _full_spec((D, HP)),
        ],
        out_specs=[_row_spec(_R), _row_spec(_R), _row_spec(_R)],
        out_shape=[jax.ShapeDtypeStruct((N, HP), jnp.float32)] * 3,
    )(x, d0, d1, w1p, w0p)


def _tc2(za, a0, a1, di, w1p, w0p, bap):
    def body(za_ref, a0_ref, a1_ref, di_ref, w1_ref, w0_ref, b_ref,
             tb_ref, zb_ref):
        dib = di_ref[...]
        h = jnp.maximum(
            za_ref[...] - dib * (a0_ref[...] + a1_ref[...]) + b_ref[0:1, :],
            0.0)
        tb_ref[...] = dib * jnp.dot(h, w1_ref[...],
                                    preferred_element_type=jnp.float32)
        zb_ref[...] = jnp.dot(h, w0_ref[...],
                              preferred_element_type=jnp.float32)

    return pl.pallas_call(
        body,
        grid=(N // _R,),
        in_specs=[
            _row_spec(_R), _row_spec(_R), _row_spec(_R), _row_spec(_R),
            _full_spec((HP, HP)), _full_spec((HP, HP)), _full_spec((8, HP)),
        ],
        out_specs=[_row_spec(_R), _row_spec(_R)],
        out_shape=[jax.ShapeDtypeStruct((N, HP), jnp.float32)] * 2,
    )(za, a0, a1, di, w1p, w0p, bap)


def _tc3(zb, a0, a1, di, wop, bbp, bop):
    def body(zb_ref, a0_ref, a1_ref, di_ref, wo_ref, bb_ref, bo_ref, o_ref):
        h = jnp.maximum(
            zb_ref[...] - di_ref[...] * (a0_ref[...] + a1_ref[...])
            + bb_ref[0:1, :],
            0.0)
        o_ref[...] = (jnp.dot(h, wo_ref[...],
                              preferred_element_type=jnp.float32)
                      + bo_ref[0:1, :])

    return pl.pallas_call(
        body,
        grid=(N // _R,),
        in_specs=[
            _row_spec(_R), _row_spec(_R), _row_spec(_R), _row_spec(_R),
            _full_spec((HP, HP)), _full_spec((8, HP)), _full_spec((8, HP)),
        ],
        out_specs=_row_spec(_R),
        out_shape=jax.ShapeDtypeStruct((N, HP), jnp.float32),
    )(zb, a0, a1, di, wop, bbp, bop)


def kernel(x, edge_index, W0a, W1a, ba, W0b, W1b, bb, Wo, bo):
    H = W0a.shape[1]
    C = Wo.shape[1]
    src = edge_index[0].reshape(NW, G, K)
    dst = edge_index[1].reshape(NW, G, K)

    w0a = jnp.pad(W0a, ((0, 0), (0, HP - H)))
    w1a = jnp.pad(W1a, ((0, 0), (0, HP - H)))
    w0b = jnp.pad(W0b, ((0, HP - H), (0, HP - H)))
    w1b = jnp.pad(W1b, ((0, HP - H), (0, HP - H)))
    wo = jnp.pad(Wo, ((0, HP - H), (0, HP - C)))
    bap = jnp.tile(jnp.pad(ba, (0, HP - H)).reshape(1, HP), (8, 1))
    bbp = jnp.tile(jnp.pad(bb, (0, HP - H)).reshape(1, HP), (8, 1))
    bop = jnp.tile(jnp.pad(bo, (0, HP - C)).reshape(1, HP), (8, 1))

    degp = _sc_degree(src)
    ta, za, di = _tc1(x, degp[0], degp[1], w1a, w0a)
    aggp = _sc_gather_scatter(ta, src, dst)
    tb, zb = _tc2(za, aggp[0], aggp[1], di, w1b, w0b, bap)
    aggp2 = _sc_gather_scatter(tb, src, dst)
    out16 = _tc3(zb, aggp2[0], aggp2[1], di, wo, bbp, bop)
    return out16[:, :C]


# speedup vs baseline: 34.3981x; 1.3382x over previous
import functools

import jax
import jax.numpy as jnp
from jax import lax
from jax.experimental import pallas as pl
from jax.experimental.pallas import tpu as pltpu
from jax.experimental.pallas import tpu_sc as plsc

N = 10000
E = 320000
D = 128
HP = 16
NC = 2
NS = 16
NW = NC * NS
EW = E // NW
G = 100
K = EW // G
RT = N // NS
RTC = 632


def _tile_row_base(s):
    return pl.multiple_of((s * RT) // 8 * 8, 8)

_mesh = plsc.VectorSubcoreMesh(core_axis_name="c", subcore_axis_name="s")


@functools.partial(
    pl.kernel,
    out_type=jax.ShapeDtypeStruct((NC, N, HP), jnp.float32),
    mesh=_mesh,
    compiler_params=pltpu.CompilerParams(use_tc_tiling_on_sc=False),
    scratch_types=[
        pltpu.VMEM((G, K), jnp.int32),
        pltpu.VMEM((K, HP), jnp.float32),
        pltpu.VMEM((RTC, HP), jnp.float32),
        pltpu.VMEM_SHARED((N, HP), jnp.float32),
    ],
)
def _sc_degree(src_hbm, deg_out, src_v, ones_v, zbuf, acc):
    c = lax.axis_index("c")
    s = lax.axis_index("s")
    wid = s * NC + c

    pltpu.sync_copy(src_hbm.at[wid], src_v)

    def _ones_row(i, carry):
        ones_v[i] = jnp.ones((HP,), jnp.float32)
        return carry

    lax.fori_loop(0, K, _ones_row, 0)

    def _zero_row(i, carry):
        zbuf[i] = jnp.zeros((HP,), jnp.float32)
        return carry

    lax.fori_loop(0, RTC, _zero_row, 0)
    base = _tile_row_base(s)
    pltpu.sync_copy(zbuf, acc.at[pl.ds(base, RTC)])
    plsc.subcore_barrier()

    def _scat(g, carry):
        pltpu.sync_copy(ones_v, acc.at[src_v.at[g]], add=True)
        return carry

    lax.fori_loop(0, G, _scat, 0)
    plsc.subcore_barrier()
    pltpu.sync_copy(acc.at[pl.ds(base, RTC)], deg_out.at[c, pl.ds(base, RTC)])


@functools.partial(
    pl.kernel,
    out_type=jax.ShapeDtypeStruct((NC, N, HP), jnp.float32),
    mesh=_mesh,
    compiler_params=pltpu.CompilerParams(use_tc_tiling_on_sc=False),
    scratch_types=[
        pltpu.VMEM((G, K), jnp.int32),
        pltpu.VMEM((G, K), jnp.int32),
        pltpu.VMEM((K, HP), jnp.float32),
        pltpu.VMEM((K, HP), jnp.float32),
        pltpu.VMEM((RTC, HP), jnp.float32),
        pltpu.VMEM_SHARED((N, HP), jnp.float32),
        pltpu.SemaphoreType.DMA,
        pltpu.SemaphoreType.DMA,
    ],
)
def _sc_gather_scatter(table_hbm, src_hbm, dst_hbm, out_hbm,
                       src_v, dst_v, rows0, rows1, zbuf, acc, sem0, sem1):
    c = lax.axis_index("c")
    s = lax.axis_index("s")
    wid = s * NC + c

    pltpu.sync_copy(src_hbm.at[wid], src_v)
    pltpu.sync_copy(dst_hbm.at[wid], dst_v)

    def _zero_row(i, carry):
        zbuf[i] = jnp.zeros((HP,), jnp.float32)
        return carry

    lax.fori_loop(0, RTC, _zero_row, 0)
    base = _tile_row_base(s)
    pltpu.sync_copy(zbuf, acc.at[pl.ds(base, RTC)])
    plsc.subcore_barrier()

    bufs = (rows0, rows1)
    sems = (sem0, sem1)
    for b in range(2):
        pltpu.async_copy(table_hbm.at[src_v.at[b]], bufs[b], sems[b])

    def _edge_pair(i, carry):
        o = 2 * i
        for b in range(2):
            g = o + b
            pltpu.make_async_copy(table_hbm.at[src_v.at[g]],
                                  bufs[b], sems[b]).wait()
            pltpu.sync_copy(bufs[b], acc.at[dst_v.at[g]], add=True)

            @pl.when(g + 2 < G)
            def _():
                pltpu.async_copy(table_hbm.at[src_v.at[g + 2]],
                                 bufs[b], sems[b])
        return carry

    lax.fori_loop(0, G // 2, _edge_pair, 0)
    plsc.subcore_barrier()
    pltpu.sync_copy(acc.at[pl.ds(base, RTC)], out_hbm.at[c, pl.ds(base, RTC)])


def _row_spec(r):
    return pl.BlockSpec((r, HP), lambda i: (i, 0))


def _full_spec(shape):
    return pl.BlockSpec(shape, lambda i: tuple(0 for _ in shape))


_R = 2000


def _tc_mm(x, w1p, w0p):
    def body(x_ref, w1_ref, w0_ref, ya_ref, za_ref):
        xb = x_ref[...]
        ya_ref[...] = jnp.dot(xb, w1_ref[...],
                              preferred_element_type=jnp.float32)
        za_ref[...] = jnp.dot(xb, w0_ref[...],
                              preferred_element_type=jnp.float32)

    return pl.pallas_call(
        body,
        grid=(N // _R,),
        in_specs=[
            pl.BlockSpec((_R, D), lambda i: (i, 0)),
            _full_spec((D, HP)), _full_spec((D, HP)),
        ],
        out_specs=[_row_spec(_R), _row_spec(_R)],
        out_shape=[jax.ShapeDtypeStruct((N, HP), jnp.float32)] * 2,
    )(x, w1p, w0p)


def _tc_scale(ya, d0, d1):
    def body(ya_ref, d0_ref, d1_ref, ta_ref, di_ref):
        deg = d0_ref[:, 0:1] + d1_ref[:, 0:1]
        dinv = jnp.where(deg > 0, lax.rsqrt(jnp.maximum(deg, 1e-12)), 0.0)
        ta_ref[...] = dinv * ya_ref[...]
        di_ref[...] = jnp.broadcast_to(dinv, (_R, HP))

    return pl.pallas_call(
        body,
        grid=(N // _R,),
        in_specs=[_row_spec(_R), _row_spec(_R), _row_spec(_R)],
        out_specs=[_row_spec(_R), _row_spec(_R)],
        out_shape=[jax.ShapeDtypeStruct((N, HP), jnp.float32)] * 2,
    )(ya, d0, d1)


def _tc2(za, a0, a1, di, w1p, w0p, bap):
    def body(za_ref, a0_ref, a1_ref, di_ref, w1_ref, w0_ref, b_ref,
             tb_ref, zb_ref):
        dib = di_ref[...]
        h = jnp.maximum(
            za_ref[...] - dib * (a0_ref[...] + a1_ref[...]) + b_ref[0:1, :],
            0.0)
        tb_ref[...] = dib * jnp.dot(h, w1_ref[...],
                                    preferred_element_type=jnp.float32)
        zb_ref[...] = jnp.dot(h, w0_ref[...],
                              preferred_element_type=jnp.float32)

    return pl.pallas_call(
        body,
        grid=(N // _R,),
        in_specs=[
            _row_spec(_R), _row_spec(_R), _row_spec(_R), _row_spec(_R),
            _full_spec((HP, HP)), _full_spec((HP, HP)), _full_spec((8, HP)),
        ],
        out_specs=[_row_spec(_R), _row_spec(_R)],
        out_shape=[jax.ShapeDtypeStruct((N, HP), jnp.float32)] * 2,
    )(za, a0, a1, di, w1p, w0p, bap)


def _tc3(zb, a0, a1, di, wop, bbp, bop):
    def body(zb_ref, a0_ref, a1_ref, di_ref, wo_ref, bb_ref, bo_ref, o_ref):
        h = jnp.maximum(
            zb_ref[...] - di_ref[...] * (a0_ref[...] + a1_ref[...])
            + bb_ref[0:1, :],
            0.0)
        o_ref[...] = (jnp.dot(h, wo_ref[...],
                              preferred_element_type=jnp.float32)
                      + bo_ref[0:1, :])

    return pl.pallas_call(
        body,
        grid=(N // _R,),
        in_specs=[
            _row_spec(_R), _row_spec(_R), _row_spec(_R), _row_spec(_R),
            _full_spec((HP, HP)), _full_spec((8, HP)), _full_spec((8, HP)),
        ],
        out_specs=_row_spec(_R),
        out_shape=jax.ShapeDtypeStruct((N, HP), jnp.float32),
    )(zb, a0, a1, di, wop, bbp, bop)


def kernel(x, edge_index, W0a, W1a, ba, W0b, W1b, bb, Wo, bo):
    H = W0a.shape[1]
    C = Wo.shape[1]
    src = edge_index[0].reshape(NW, G, K)
    dst = edge_index[1].reshape(NW, G, K)

    w0a = jnp.pad(W0a, ((0, 0), (0, HP - H)))
    w1a = jnp.pad(W1a, ((0, 0), (0, HP - H)))
    w0b = jnp.pad(W0b, ((0, HP - H), (0, HP - H)))
    w1b = jnp.pad(W1b, ((0, HP - H), (0, HP - H)))
    wo = jnp.pad(Wo, ((0, HP - H), (0, HP - C)))
    bap = jnp.tile(jnp.pad(ba, (0, HP - H)).reshape(1, HP), (8, 1))
    bbp = jnp.tile(jnp.pad(bb, (0, HP - H)).reshape(1, HP), (8, 1))
    bop = jnp.tile(jnp.pad(bo, (0, HP - C)).reshape(1, HP), (8, 1))

    degp = _sc_degree(src)
    ya, za = _tc_mm(x, w1a, w0a)
    ta, di = _tc_scale(ya, degp[0], degp[1])
    aggp = _sc_gather_scatter(ta, src, dst)
    tb, zb = _tc2(za, aggp[0], aggp[1], di, w1b, w0b, bap)
    aggp2 = _sc_gather_scatter(tb, src, dst)
    out16 = _tc3(zb, aggp2[0], aggp2[1], di, wo, bbp, bop)
    return out16[:, :C]


# speedup vs baseline: 55.2468x; 1.6061x over previous
import functools

import jax
import jax.numpy as jnp
from jax import lax
from jax.experimental import pallas as pl
from jax.experimental.pallas import tpu as pltpu
from jax.experimental.pallas import tpu_sc as plsc

N = 10000
E = 320000
D = 128
HP = 16
NC = 2
NS = 16
NW = NC * NS
EW = E // NW
G = 80
K = EW // G
RT = N // NS
RTC = 632
ZR = 80
NPAD = 10016
NB = 4


def _tile_row_base(s):
    return pl.multiple_of((s * RT) // 8 * 8, 8)


def _zero_acc(zbuf, acc, base, zsem):
    def _zero_row(i, carry):
        zbuf[i] = jnp.zeros((HP,), jnp.float32)
        return carry

    lax.fori_loop(0, ZR, _zero_row, 0)
    for k in range(8):
        pltpu.async_copy(zbuf, acc.at[pl.ds(base + ZR * k, ZR)], zsem)
    for k in range(8):
        pltpu.make_async_copy(zbuf, acc.at[pl.ds(base + ZR * k, ZR)],
                              zsem).wait()


_mesh = plsc.VectorSubcoreMesh(core_axis_name="c", subcore_axis_name="s")


@functools.partial(
    pl.kernel,
    out_type=jax.ShapeDtypeStruct((NC, N, HP), jnp.float32),
    mesh=_mesh,
    compiler_params=pltpu.CompilerParams(use_tc_tiling_on_sc=False),
    scratch_types=[
        pltpu.VMEM((G, K), jnp.int32),
        pltpu.VMEM((K, HP), jnp.float32),
        pltpu.VMEM((ZR, HP), jnp.float32),
        pltpu.VMEM_SHARED((NPAD, HP), jnp.float32),
        [pltpu.SemaphoreType.DMA] * NB,
        pltpu.SemaphoreType.DMA,
    ],
)
def _sc_degree(src_hbm, deg_out, src_v, ones_v, zbuf, acc, ss, zsem):
    c = lax.axis_index("c")
    s = lax.axis_index("s")
    wid = s * NC + c

    pltpu.sync_copy(src_hbm.at[wid], src_v)

    def _ones_row(i, carry):
        ones_v[i] = jnp.ones((HP,), jnp.float32)
        return carry

    lax.fori_loop(0, K, _ones_row, 0)
    base = _tile_row_base(s)
    _zero_acc(zbuf, acc, base, zsem)
    plsc.subcore_barrier()

    def _scat4(i, carry):
        for j in range(NB):
            g = NB * i + j

            @pl.when(g >= NB)
            def _():
                pltpu.make_async_copy(ones_v, acc.at[src_v.at[g - NB]],
                                      ss[j]).wait()

            pltpu.async_copy(ones_v, acc.at[src_v.at[g]], ss[j], add=True)
        return carry

    lax.fori_loop(0, G // NB, _scat4, 0)
    for j in range(NB):
        pltpu.make_async_copy(ones_v, acc.at[src_v.at[G - NB + j]],
                              ss[j]).wait()
    plsc.subcore_barrier()
    pltpu.sync_copy(acc.at[pl.ds(base, RTC)], deg_out.at[c, pl.ds(base, RTC)])


@functools.partial(
    pl.kernel,
    out_type=jax.ShapeDtypeStruct((NC, N, HP), jnp.float32),
    mesh=_mesh,
    compiler_params=pltpu.CompilerParams(use_tc_tiling_on_sc=False),
    scratch_types=[
        pltpu.VMEM((G, K), jnp.int32),
        pltpu.VMEM((G, K), jnp.int32),
        [pltpu.VMEM((K, HP), jnp.float32)] * NB,
        pltpu.VMEM((ZR, HP), jnp.float32),
        pltpu.VMEM_SHARED((NPAD, HP), jnp.float32),
        [pltpu.SemaphoreType.DMA] * NB,
        [pltpu.SemaphoreType.DMA] * NB,
        pltpu.SemaphoreType.DMA,
    ],
)
def _sc_gather_scatter(table_hbm, src_hbm, dst_hbm, out_hbm,
                       src_v, dst_v, bufs, zbuf, acc, gs, ss, zsem):
    c = lax.axis_index("c")
    s = lax.axis_index("s")
    wid = s * NC + c

    pltpu.sync_copy(src_hbm.at[wid], src_v)
    pltpu.sync_copy(dst_hbm.at[wid], dst_v)
    base = _tile_row_base(s)
    _zero_acc(zbuf, acc, base, zsem)
    plsc.subcore_barrier()

    for b in range(2):
        pltpu.async_copy(table_hbm.at[src_v.at[b]], bufs[b], gs[b])

    def _pipe4(i, carry):
        for j in range(NB):
            g = NB * i + j
            b = j
            b2 = (j + 2) % NB
            pltpu.make_async_copy(table_hbm.at[src_v.at[g]],
                                  bufs[b], gs[b]).wait()
            pltpu.async_copy(bufs[b], acc.at[dst_v.at[g]], ss[b], add=True)

            @pl.when(g >= 2)
            def _():
                pltpu.make_async_copy(bufs[b2], acc.at[dst_v.at[g - 2]],
                                      ss[b2]).wait()

            @pl.when(g + 2 < G)
            def _():
                pltpu.async_copy(table_hbm.at[src_v.at[g + 2]],
                                 bufs[b2], gs[b2])
        return carry

    lax.fori_loop(0, G // NB, _pipe4, 0)
    for g in (G - 2, G - 1):
        pltpu.make_async_copy(bufs[g % NB], acc.at[dst_v.at[g]],
                              ss[g % NB]).wait()
    plsc.subcore_barrier()
    pltpu.sync_copy(acc.at[pl.ds(base, RTC)], out_hbm.at[c, pl.ds(base, RTC)])


N8 = N // 8
LW = 128


def _pk_spec(r):
    return pl.BlockSpec((r, LW), lambda i: (i, 0))


def _part_spec(core):
    return pl.BlockSpec((1, N8, LW), lambda i, core=core: (core, i, 0))


def _full_spec(shape):
    return pl.BlockSpec(shape, lambda i: tuple(0 for _ in shape))


_R8 = N8


def _tc_mm(x_pk, w1blk, w0blk):
    def body(x_ref, w1_ref, w0_ref, ya_ref, za_ref):
        xb = x_ref[...]
        ya_ref[...] = jnp.dot(xb, w1_ref[...],
                              preferred_element_type=jnp.float32)
        za_ref[...] = jnp.dot(xb, w0_ref[...],
                              preferred_element_type=jnp.float32)

    return pl.pallas_call(
        body,
        grid=(1,),
        in_specs=[
            pl.BlockSpec((_R8, 8 * D), lambda i: (i, 0)),
            _full_spec((8 * D, LW)), _full_spec((8 * D, LW)),
        ],
        out_specs=[_pk_spec(_R8), _pk_spec(_R8)],
        out_shape=[jax.ShapeDtypeStruct((N8, LW), jnp.float32)] * 2,
    )(x_pk, w1blk, w0blk)


def _tc_scale(ya, degp):
    def body(ya_ref, d0_ref, d1_ref, ta_ref, di_ref):
        deg = d0_ref[0] + d1_ref[0]
        dinv = jnp.where(deg > 0, lax.rsqrt(jnp.maximum(deg, 1e-12)), 0.0)
        ta_ref[...] = dinv * ya_ref[...]
        di_ref[...] = dinv

    return pl.pallas_call(
        body,
        grid=(1,),
        in_specs=[_pk_spec(_R8), _part_spec(0), _part_spec(1)],
        out_specs=[_pk_spec(_R8), _pk_spec(_R8)],
        out_shape=[jax.ShapeDtypeStruct((N8, LW), jnp.float32)] * 2,
    )(ya, degp, degp)


def _tc2(za, aggp, di, w1blk, w0blk, bat):
    def body(za_ref, a0_ref, a1_ref, di_ref, w1_ref, w0_ref, b_ref,
             tb_ref, zb_ref):
        dib = di_ref[...]
        h = jnp.maximum(
            za_ref[...] - dib * (a0_ref[0] + a1_ref[0]) + b_ref[0:1, :],
            0.0)
        tb_ref[...] = dib * jnp.dot(h, w1_ref[...],
                                    preferred_element_type=jnp.float32)
        zb_ref[...] = jnp.dot(h, w0_ref[...],
                              preferred_element_type=jnp.float32)

    return pl.pallas_call(
        body,
        grid=(1,),
        in_specs=[
            _pk_spec(_R8), _part_spec(0), _part_spec(1), _pk_spec(_R8),
            _full_spec((LW, LW)), _full_spec((LW, LW)), _full_spec((8, LW)),
        ],
        out_specs=[_pk_spec(_R8), _pk_spec(_R8)],
        out_shape=[jax.ShapeDtypeStruct((N8, LW), jnp.float32)] * 2,
    )(za, aggp, aggp, di, w1blk, w0blk, bat)


def _tc3(zb, aggp, di, woblk, bbt, bot):
    def body(zb_ref, a0_ref, a1_ref, di_ref, wo_ref, bb_ref, bo_ref, o_ref):
        h = jnp.maximum(
            zb_ref[...] - di_ref[...] * (a0_ref[0] + a1_ref[0])
            + bb_ref[0:1, :],
            0.0)
        o_ref[...] = (jnp.dot(h, wo_ref[...],
                              preferred_element_type=jnp.float32)
                      + bo_ref[0:1, :])

    return pl.pallas_call(
        body,
        grid=(1,),
        in_specs=[
            _pk_spec(_R8), _part_spec(0), _part_spec(1), _pk_spec(_R8),
            _full_spec((LW, LW)), _full_spec((8, LW)), _full_spec((8, LW)),
        ],
        out_specs=_pk_spec(_R8),
        out_shape=jax.ShapeDtypeStruct((N8, LW), jnp.float32),
    )(zb, aggp, aggp, di, woblk, bbt, bot)


def kernel(x, edge_index, W0a, W1a, ba, W0b, W1b, bb, Wo, bo):
    H = W0a.shape[1]
    C = Wo.shape[1]
    src = edge_index[0].reshape(NW, G, K)
    dst = edge_index[1].reshape(NW, G, K)

    eye8 = jnp.eye(8, dtype=jnp.float32)
    w1a_blk = jnp.kron(eye8, jnp.pad(W1a, ((0, 0), (0, HP - H))))
    w0a_blk = jnp.kron(eye8, jnp.pad(W0a, ((0, 0), (0, HP - H))))
    w1b_blk = jnp.kron(eye8, jnp.pad(W1b, ((0, HP - H), (0, HP - H))))
    w0b_blk = jnp.kron(eye8, jnp.pad(W0b, ((0, HP - H), (0, HP - H))))
    wo_blk = jnp.kron(eye8, jnp.pad(Wo, ((0, HP - H), (0, HP - C))))
    bat = jnp.tile(jnp.pad(ba, (0, HP - H)), (8, 8))
    bbt = jnp.tile(jnp.pad(bb, (0, HP - H)), (8, 8))
    bot = jnp.tile(jnp.pad(bo, (0, HP - C)), (8, 8))

    x_pk = x.reshape(N8, 8 * D)
    deg_pk = _sc_degree(src).reshape(NC, N8, LW)
    ya, za = _tc_mm(x_pk, w1a_blk, w0a_blk)
    ta_pk, di = _tc_scale(ya, deg_pk)
    agg_pk = _sc_gather_scatter(ta_pk.reshape(N, HP), src,
                                dst).reshape(NC, N8, LW)
    tb_pk, zb = _tc2(za, agg_pk, di, w1b_blk, w0b_blk, bat)
    agg2_pk = _sc_gather_scatter(tb_pk.reshape(N, HP), src,
                                 dst).reshape(NC, N8, LW)
    o_pk = _tc3(zb, agg2_pk, di, wo_blk, bbt, bot)
    return o_pk.reshape(N, HP)[:, :C]


# speedup vs baseline: 67.7321x; 1.2260x over previous
import functools

import jax
import jax.numpy as jnp
from jax import lax
from jax.experimental import pallas as pl
from jax.experimental.pallas import tpu as pltpu
from jax.experimental.pallas import tpu_sc as plsc

N = 10000
E = 320000
D = 128
HP = 16
NC = 2
NS = 16
NW = NC * NS
EW = E // NW
G = 80
K = EW // G
RT = N // NS
RTC = 632
ZR = 80
NPAD = 10016
NB = 8
QD = 4


def _tile_row_base(s):
    return pl.multiple_of((s * RT) // 8 * 8, 8)


def _zero_acc(zbuf, acc, base, zsem):
    def _zero_row(i, carry):
        zbuf[i] = jnp.zeros((HP,), jnp.float32)
        return carry

    lax.fori_loop(0, ZR, _zero_row, 0)
    for k in range(8):
        pltpu.async_copy(zbuf, acc.at[pl.ds(base + ZR * k, ZR)], zsem)
    for k in range(8):
        pltpu.make_async_copy(zbuf, acc.at[pl.ds(base + ZR * k, ZR)],
                              zsem).wait()


_mesh = plsc.VectorSubcoreMesh(core_axis_name="c", subcore_axis_name="s")


@functools.partial(
    pl.kernel,
    out_type=jax.ShapeDtypeStruct((NC, N, HP), jnp.float32),
    mesh=_mesh,
    compiler_params=pltpu.CompilerParams(use_tc_tiling_on_sc=False),
    scratch_types=[
        pltpu.VMEM((G, K), jnp.int32),
        pltpu.VMEM((K, HP), jnp.float32),
        pltpu.VMEM((ZR, HP), jnp.float32),
        pltpu.VMEM_SHARED((NPAD, HP), jnp.float32),
        [pltpu.SemaphoreType.DMA] * NB,
        pltpu.SemaphoreType.DMA,
    ],
)
def _sc_degree(src_hbm, deg_out, src_v, ones_v, zbuf, acc, ss, zsem):
    c = lax.axis_index("c")
    s = lax.axis_index("s")
    wid = s * NC + c

    pltpu.sync_copy(src_hbm.at[wid], src_v)

    def _ones_row(i, carry):
        ones_v[i] = jnp.ones((HP,), jnp.float32)
        return carry

    lax.fori_loop(0, K, _ones_row, 0)
    base = _tile_row_base(s)
    _zero_acc(zbuf, acc, base, zsem)
    plsc.subcore_barrier()

    def _scat4(i, carry):
        for j in range(NB):
            g = NB * i + j

            @pl.when(g >= NB)
            def _():
                pltpu.make_async_copy(ones_v, acc.at[src_v.at[g - NB]],
                                      ss[j]).wait()

            pltpu.async_copy(ones_v, acc.at[src_v.at[g]], ss[j], add=True)
        return carry

    lax.fori_loop(0, G // NB, _scat4, 0)
    for j in range(NB):
        pltpu.make_async_copy(ones_v, acc.at[src_v.at[G - NB + j]],
                              ss[j]).wait()
    plsc.subcore_barrier()
    pltpu.sync_copy(acc.at[pl.ds(base, RTC)], deg_out.at[c, pl.ds(base, RTC)])


@functools.partial(
    pl.kernel,
    out_type=jax.ShapeDtypeStruct((NC, N, HP), jnp.float32),
    mesh=_mesh,
    compiler_params=pltpu.CompilerParams(use_tc_tiling_on_sc=False),
    scratch_types=[
        pltpu.VMEM((G, K), jnp.int32),
        pltpu.VMEM((G, K), jnp.int32),
        [pltpu.VMEM((K, HP), jnp.float32)] * NB,
        pltpu.VMEM((ZR, HP), jnp.float32),
        pltpu.VMEM_SHARED((NPAD, HP), jnp.float32),
        [pltpu.SemaphoreType.DMA] * NB,
        [pltpu.SemaphoreType.DMA] * NB,
        pltpu.SemaphoreType.DMA,
    ],
)
def _sc_gather_scatter(table_hbm, src_hbm, dst_hbm, out_hbm,
                       src_v, dst_v, bufs, zbuf, acc, gs, ss, zsem):
    c = lax.axis_index("c")
    s = lax.axis_index("s")
    wid = s * NC + c

    pltpu.sync_copy(src_hbm.at[wid], src_v)
    pltpu.sync_copy(dst_hbm.at[wid], dst_v)
    base = _tile_row_base(s)
    _zero_acc(zbuf, acc, base, zsem)
    plsc.subcore_barrier()

    for b in range(QD):
        pltpu.async_copy(table_hbm.at[src_v.at[b]], bufs[b], gs[b])

    def _pipe(i, carry):
        for j in range(NB):
            g = NB * i + j
            b = j
            b2 = (j + QD) % NB
            pltpu.make_async_copy(table_hbm.at[src_v.at[g]],
                                  bufs[b], gs[b]).wait()
            pltpu.async_copy(bufs[b], acc.at[dst_v.at[g]], ss[b], add=True)

            @pl.when(g >= QD)
            def _():
                pltpu.make_async_copy(bufs[b2], acc.at[dst_v.at[g - QD]],
                                      ss[b2]).wait()

            @pl.when(g + QD < G)
            def _():
                pltpu.async_copy(table_hbm.at[src_v.at[g + QD]],
                                 bufs[b2], gs[b2])
        return carry

    lax.fori_loop(0, G // NB, _pipe, 0)
    for g in range(G - QD, G):
        pltpu.make_async_copy(bufs[g % NB], acc.at[dst_v.at[g]],
                              ss[g % NB]).wait()
    plsc.subcore_barrier()
    pltpu.sync_copy(acc.at[pl.ds(base, RTC)], out_hbm.at[c, pl.ds(base, RTC)])


N8 = N // 8
LW = 128


def _pk_spec(r):
    return pl.BlockSpec((r, LW), lambda i: (i, 0))


def _part_spec(core):
    return pl.BlockSpec((1, N8, LW), lambda i, core=core: (core, i, 0))


def _full_spec(shape):
    return pl.BlockSpec(shape, lambda i: tuple(0 for _ in shape))


_R8 = N8


def _tc_mm(x_pk, wblk):
    def body(x_ref, w_ref, y_ref):
        y_ref[...] = jnp.dot(x_ref[...], w_ref[...],
                             preferred_element_type=jnp.float32)

    return pl.pallas_call(
        body,
        grid=(1,),
        in_specs=[
            pl.BlockSpec((_R8, 8 * D), lambda i: (i, 0)),
            _full_spec((8 * D, LW)),
        ],
        out_specs=_pk_spec(_R8),
        out_shape=jax.ShapeDtypeStruct((N8, LW), jnp.float32),
    )(x_pk, wblk)


def _tc_scale(ya, degp):
    def body(ya_ref, d0_ref, d1_ref, ta_ref, di_ref):
        deg = d0_ref[0] + d1_ref[0]
        dinv = jnp.where(deg > 0, lax.rsqrt(jnp.maximum(deg, 1e-12)), 0.0)
        ta_ref[...] = dinv * ya_ref[...]
        di_ref[...] = dinv

    return pl.pallas_call(
        body,
        grid=(1,),
        in_specs=[_pk_spec(_R8), _part_spec(0), _part_spec(1)],
        out_specs=[_pk_spec(_R8), _pk_spec(_R8)],
        out_shape=[jax.ShapeDtypeStruct((N8, LW), jnp.float32)] * 2,
    )(ya, degp, degp)


def _tc2(za, aggp, di, w1blk, w0blk, bat):
    def body(za_ref, a0_ref, a1_ref, di_ref, w1_ref, w0_ref, b_ref,
             tb_ref, zb_ref):
        dib = di_ref[...]
        h = jnp.maximum(
            za_ref[...] - dib * (a0_ref[0] + a1_ref[0]) + b_ref[0:1, :],
            0.0)
        tb_ref[...] = dib * jnp.dot(h, w1_ref[...],
                                    preferred_element_type=jnp.float32)
        zb_ref[...] = jnp.dot(h, w0_ref[...],
                              preferred_element_type=jnp.float32)

    return pl.pallas_call(
        body,
        grid=(1,),
        in_specs=[
            _pk_spec(_R8), _part_spec(0), _part_spec(1), _pk_spec(_R8),
            _full_spec((LW, LW)), _full_spec((LW, LW)), _full_spec((8, LW)),
        ],
        out_specs=[_pk_spec(_R8), _pk_spec(_R8)],
        out_shape=[jax.ShapeDtypeStruct((N8, LW), jnp.float32)] * 2,
    )(za, aggp, aggp, di, w1blk, w0blk, bat)


def _tc3(zb, aggp, di, woblk, bbt, bot):
    def body(zb_ref, a0_ref, a1_ref, di_ref, wo_ref, bb_ref, bo_ref, o_ref):
        h = jnp.maximum(
            zb_ref[...] - di_ref[...] * (a0_ref[0] + a1_ref[0])
            + bb_ref[0:1, :],
            0.0)
        o_ref[...] = (jnp.dot(h, wo_ref[...],
                              preferred_element_type=jnp.float32)
                      + bo_ref[0:1, :])

    return pl.pallas_call(
        body,
        grid=(1,),
        in_specs=[
            _pk_spec(_R8), _part_spec(0), _part_spec(1), _pk_spec(_R8),
            _full_spec((LW, LW)), _full_spec((8, LW)), _full_spec((8, LW)),
        ],
        out_specs=_pk_spec(_R8),
        out_shape=jax.ShapeDtypeStruct((N8, LW), jnp.float32),
    )(zb, aggp, aggp, di, woblk, bbt, bot)


def kernel(x, edge_index, W0a, W1a, ba, W0b, W1b, bb, Wo, bo):
    H = W0a.shape[1]
    C = Wo.shape[1]
    src = edge_index[0].reshape(NW, G, K)
    dst = edge_index[1].reshape(NW, G, K)

    eye8 = jnp.eye(8, dtype=jnp.float32)
    w1a_blk = jnp.kron(eye8, jnp.pad(W1a, ((0, 0), (0, HP - H))))
    w0a_blk = jnp.kron(eye8, jnp.pad(W0a, ((0, 0), (0, HP - H))))
    w1b_blk = jnp.kron(eye8, jnp.pad(W1b, ((0, HP - H), (0, HP - H))))
    w0b_blk = jnp.kron(eye8, jnp.pad(W0b, ((0, HP - H), (0, HP - H))))
    wo_blk = jnp.kron(eye8, jnp.pad(Wo, ((0, HP - H), (0, HP - C))))
    bat = jnp.tile(jnp.pad(ba, (0, HP - H)), (8, 8))
    bbt = jnp.tile(jnp.pad(bb, (0, HP - H)), (8, 8))
    bot = jnp.tile(jnp.pad(bo, (0, HP - C)), (8, 8))

    x_pk = x.reshape(N8, 8 * D)
    deg_pk = _sc_degree(src).reshape(NC, N8, LW)
    ya = _tc_mm(x_pk, w1a_blk)
    ta_pk, di = _tc_scale(ya, deg_pk)
    za = _tc_mm(x_pk, w0a_blk)
    agg_pk = _sc_gather_scatter(ta_pk.reshape(N, HP), src,
                                dst).reshape(NC, N8, LW)
    tb_pk, zb = _tc2(za, agg_pk, di, w1b_blk, w0b_blk, bat)
    agg2_pk = _sc_gather_scatter(tb_pk.reshape(N, HP), src,
                                 dst).reshape(NC, N8, LW)
    o_pk = _tc3(zb, agg2_pk, di, wo_blk, bbt, bot)
    return o_pk.reshape(N, HP)[:, :C]


# speedup vs baseline: 75.8111x; 1.1193x over previous
import functools

import jax
import jax.numpy as jnp
from jax import lax
from jax.experimental import pallas as pl
from jax.experimental.pallas import tpu as pltpu
from jax.experimental.pallas import tpu_sc as plsc

N = 10000
E = 320000
D = 128
HP = 16
NC = 2
NS = 16
NW = NC * NS
EW = E // NW
G = 80
K = EW // G
RT = N // NS
RTC = 632
ZR = 80
NPAD = 10016
NB = 8
QD = 4


def _tile_row_base(s):
    return pl.multiple_of((s * RT) // 8 * 8, 8)


def _zero_acc(zbuf, acc, base, zsem):
    def _zero_row(i, carry):
        zbuf[i] = jnp.zeros((HP,), jnp.float32)
        return carry

    lax.fori_loop(0, ZR, _zero_row, 0)
    for k in range(8):
        pltpu.async_copy(zbuf, acc.at[pl.ds(base + ZR * k, ZR)], zsem)
    for k in range(8):
        pltpu.make_async_copy(zbuf, acc.at[pl.ds(base + ZR * k, ZR)],
                              zsem).wait()


_mesh = plsc.VectorSubcoreMesh(core_axis_name="c", subcore_axis_name="s")


@functools.partial(
    pl.kernel,
    out_type=jax.ShapeDtypeStruct((NC, N, HP), jnp.float32),
    mesh=_mesh,
    compiler_params=pltpu.CompilerParams(use_tc_tiling_on_sc=False),
    scratch_types=[
        pltpu.VMEM((G, K), jnp.int32),
        pltpu.VMEM((K, HP), jnp.float32),
        pltpu.VMEM((ZR, HP), jnp.float32),
        pltpu.VMEM_SHARED((NPAD, HP), jnp.float32),
        [pltpu.SemaphoreType.DMA] * NB,
        pltpu.SemaphoreType.DMA,
    ],
)
def _sc_degree(src_hbm, deg_out, src_v, ones_v, zbuf, acc, ss, zsem):
    c = lax.axis_index("c")
    s = lax.axis_index("s")
    wid = s * NC + c

    pltpu.sync_copy(src_hbm.at[wid], src_v)

    def _ones_row(i, carry):
        ones_v[i] = jnp.ones((HP,), jnp.float32)
        return carry

    lax.fori_loop(0, K, _ones_row, 0)
    base = _tile_row_base(s)
    _zero_acc(zbuf, acc, base, zsem)
    plsc.subcore_barrier()

    def _scat4(i, carry):
        for j in range(NB):
            g = NB * i + j

            @pl.when(g >= NB)
            def _():
                pltpu.make_async_copy(ones_v, acc.at[src_v.at[g - NB]],
                                      ss[j]).wait()

            pltpu.async_copy(ones_v, acc.at[src_v.at[g]], ss[j], add=True)
        return carry

    lax.fori_loop(0, G // NB, _scat4, 0)
    for j in range(NB):
        pltpu.make_async_copy(ones_v, acc.at[src_v.at[G - NB + j]],
                              ss[j]).wait()
    plsc.subcore_barrier()
    pltpu.sync_copy(acc.at[pl.ds(base, RTC)], deg_out.at[c, pl.ds(base, RTC)])


@functools.partial(
    pl.kernel,
    out_type=jax.ShapeDtypeStruct((NC, N, HP), jnp.float32),
    mesh=_mesh,
    compiler_params=pltpu.CompilerParams(use_tc_tiling_on_sc=False),
    scratch_types=[
        pltpu.VMEM((G, K), jnp.int32),
        pltpu.VMEM((G, K), jnp.int32),
        [pltpu.VMEM((K, HP), jnp.float32)] * NB,
        pltpu.VMEM((ZR, HP), jnp.float32),
        pltpu.VMEM_SHARED((NPAD, HP), jnp.float32),
        pltpu.VMEM_SHARED((N, HP), jnp.float32),
        [pltpu.SemaphoreType.DMA] * NB,
        [pltpu.SemaphoreType.DMA] * NB,
        pltpu.SemaphoreType.DMA,
        pltpu.SemaphoreType.DMA,
    ],
)
def _sc_gather_scatter(table_hbm, src_hbm, dst_hbm, out_hbm,
                       src_v, dst_v, bufs, zbuf, acc, tbl, gs, ss, zsem, tsem):
    c = lax.axis_index("c")
    s = lax.axis_index("s")
    wid = s * NC + c

    pltpu.sync_copy(src_hbm.at[wid], src_v)
    pltpu.sync_copy(dst_hbm.at[wid], dst_v)
    base = _tile_row_base(s)
    pltpu.async_copy(table_hbm.at[pl.ds(base, RTC)],
                     tbl.at[pl.ds(base, RTC)], tsem)
    _zero_acc(zbuf, acc, base, zsem)
    pltpu.make_async_copy(table_hbm.at[pl.ds(base, RTC)],
                          tbl.at[pl.ds(base, RTC)], tsem).wait()
    plsc.subcore_barrier()

    for b in range(QD):
        pltpu.async_copy(tbl.at[src_v.at[b]], bufs[b], gs[b])

    def _pipe(i, carry):
        for j in range(NB):
            g = NB * i + j
            b = j
            b2 = (j + QD) % NB
            pltpu.make_async_copy(tbl.at[src_v.at[g]],
                                  bufs[b], gs[b]).wait()
            pltpu.async_copy(bufs[b], acc.at[dst_v.at[g]], ss[b], add=True)

            @pl.when(g >= QD)
            def _():
                pltpu.make_async_copy(bufs[b2], acc.at[dst_v.at[g - QD]],
                                      ss[b2]).wait()

            @pl.when(g + QD < G)
            def _():
                pltpu.async_copy(tbl.at[src_v.at[g + QD]],
                                 bufs[b2], gs[b2])
        return carry

    lax.fori_loop(0, G // NB, _pipe, 0)
    for g in range(G - QD, G):
        pltpu.make_async_copy(bufs[g % NB], acc.at[dst_v.at[g]],
                              ss[g % NB]).wait()
    plsc.subcore_barrier()
    pltpu.sync_copy(acc.at[pl.ds(base, RTC)], out_hbm.at[c, pl.ds(base, RTC)])


N8 = N // 8
LW = 128


def _pk_spec(r):
    return pl.BlockSpec((r, LW), lambda i: (i, 0))


def _part_spec(core):
    return pl.BlockSpec((1, N8, LW), lambda i, core=core: (core, i, 0))


def _full_spec(shape):
    return pl.BlockSpec(shape, lambda i: tuple(0 for _ in shape))


_R8 = N8


def _tc_mm(x_pk, wblk):
    def body(x_ref, w_ref, y_ref):
        y_ref[...] = jnp.dot(x_ref[...], w_ref[...],
                             preferred_element_type=jnp.float32)

    return pl.pallas_call(
        body,
        grid=(1,),
        in_specs=[
            pl.BlockSpec((_R8, 8 * D), lambda i: (i, 0)),
            _full_spec((8 * D, LW)),
        ],
        out_specs=_pk_spec(_R8),
        out_shape=jax.ShapeDtypeStruct((N8, LW), jnp.float32),
    )(x_pk, wblk)


def _tc_scale(ya, degp):
    def body(ya_ref, d0_ref, d1_ref, ta_ref, di_ref):
        deg = d0_ref[0] + d1_ref[0]
        dinv = jnp.where(deg > 0, lax.rsqrt(jnp.maximum(deg, 1e-12)), 0.0)
        ta_ref[...] = dinv * ya_ref[...]
        di_ref[...] = dinv

    return pl.pallas_call(
        body,
        grid=(1,),
        in_specs=[_pk_spec(_R8), _part_spec(0), _part_spec(1)],
        out_specs=[_pk_spec(_R8), _pk_spec(_R8)],
        out_shape=[jax.ShapeDtypeStruct((N8, LW), jnp.float32)] * 2,
    )(ya, degp, degp)


def _tc2(za, aggp, di, w1blk, w0blk, bat):
    def body(za_ref, a0_ref, a1_ref, di_ref, w1_ref, w0_ref, b_ref,
             tb_ref, zb_ref):
        dib = di_ref[...]
        h = jnp.maximum(
            za_ref[...] - dib * (a0_ref[0] + a1_ref[0]) + b_ref[0:1, :],
            0.0)
        tb_ref[...] = dib * jnp.dot(h, w1_ref[...],
                                    preferred_element_type=jnp.float32)
        zb_ref[...] = jnp.dot(h, w0_ref[...],
                              preferred_element_type=jnp.float32)

    return pl.pallas_call(
        body,
        grid=(1,),
        in_specs=[
            _pk_spec(_R8), _part_spec(0), _part_spec(1), _pk_spec(_R8),
            _full_spec((LW, LW)), _full_spec((LW, LW)), _full_spec((8, LW)),
        ],
        out_specs=[_pk_spec(_R8), _pk_spec(_R8)],
        out_shape=[jax.ShapeDtypeStruct((N8, LW), jnp.float32)] * 2,
    )(za, aggp, aggp, di, w1blk, w0blk, bat)


def _tc3(zb, aggp, di, woblk, bbt, bot):
    def body(zb_ref, a0_ref, a1_ref, di_ref, wo_ref, bb_ref, bo_ref, o_ref):
        h = jnp.maximum(
            zb_ref[...] - di_ref[...] * (a0_ref[0] + a1_ref[0])
            + bb_ref[0:1, :],
            0.0)
        o_ref[...] = (jnp.dot(h, wo_ref[...],
                              preferred_element_type=jnp.float32)
                      + bo_ref[0:1, :])

    return pl.pallas_call(
        body,
        grid=(1,),
        in_specs=[
            _pk_spec(_R8), _part_spec(0), _part_spec(1), _pk_spec(_R8),
            _full_spec((LW, LW)), _full_spec((8, LW)), _full_spec((8, LW)),
        ],
        out_specs=_pk_spec(_R8),
        out_shape=jax.ShapeDtypeStruct((N8, LW), jnp.float32),
    )(zb, aggp, aggp, di, woblk, bbt, bot)


def kernel(x, edge_index, W0a, W1a, ba, W0b, W1b, bb, Wo, bo):
    H = W0a.shape[1]
    C = Wo.shape[1]
    src = edge_index[0].reshape(NW, G, K)
    dst = edge_index[1].reshape(NW, G, K)

    eye8 = jnp.eye(8, dtype=jnp.float32)
    w1a_blk = jnp.kron(eye8, jnp.pad(W1a, ((0, 0), (0, HP - H))))
    w0a_blk = jnp.kron(eye8, jnp.pad(W0a, ((0, 0), (0, HP - H))))
    w1b_blk = jnp.kron(eye8, jnp.pad(W1b, ((0, HP - H), (0, HP - H))))
    w0b_blk = jnp.kron(eye8, jnp.pad(W0b, ((0, HP - H), (0, HP - H))))
    wo_blk = jnp.kron(eye8, jnp.pad(Wo, ((0, HP - H), (0, HP - C))))
    bat = jnp.tile(jnp.pad(ba, (0, HP - H)), (8, 8))
    bbt = jnp.tile(jnp.pad(bb, (0, HP - H)), (8, 8))
    bot = jnp.tile(jnp.pad(bo, (0, HP - C)), (8, 8))

    x_pk = x.reshape(N8, 8 * D)
    deg_pk = _sc_degree(src).reshape(NC, N8, LW)
    ya = _tc_mm(x_pk, w1a_blk)
    ta_pk, di = _tc_scale(ya, deg_pk)
    za = _tc_mm(x_pk, w0a_blk)
    agg_pk = _sc_gather_scatter(ta_pk.reshape(N, HP), src,
                                dst).reshape(NC, N8, LW)
    tb_pk, zb = _tc2(za, agg_pk, di, w1b_blk, w0b_blk, bat)
    agg2_pk = _sc_gather_scatter(tb_pk.reshape(N, HP), src,
                                 dst).reshape(NC, N8, LW)
    o_pk = _tc3(zb, agg2_pk, di, wo_blk, bbt, bot)
    return o_pk.reshape(N, HP)[:, :C]
